# baseline (speedup 1.0000x reference)
import functools

import jax
import jax.numpy as jnp
from jax import lax
from jax.experimental import pallas as pl
from jax.experimental.pallas import tpu as pltpu
from jax.experimental.pallas import tpu_sc as plsc

N = 10000
E = 320000
D = 128
H = 256
NC = 2
NS = 16
CHUNK = 80
NPAD = 10240
ROWS_PER_SUB = NPAD // NS
ZCH = 128
RB = 1280

CHUNK1 = 40
NCH1 = E // (NC * NS) // CHUNK1
NCH2 = E // NS // CHUNK
NBLK1 = 10
BLK1 = NCH1 // NBLK1
CHUNK2 = 40
NCH2B = E // NS // CHUNK2
NBLK2 = 10
BLK2 = NCH2B // NBLK2


def _zero_acc_rows(zrows, acc, s, chunk=CHUNK):
    @pl.loop(0, chunk)
    def _(r):
        @pl.loop(0, D, step=16)
        def _(j):
            zrows[r, pl.ds(j, 16)] = jnp.zeros((16,), jnp.float32)

    @pl.loop(0, ROWS_PER_SUB // chunk)
    def _(j):
        pltpu.sync_copy(zrows,
                        acc.at[pl.ds(s * ROWS_PER_SUB + j * chunk, chunk)])


NBUF = 3
NBUF2 = 6


def _edge_pipeline(nch, fire_gather, wait_gather, scatter, nbuf=NBUF):
    for b in range(nbuf):
        fire_gather(b, b)

    @pl.loop(0, nch // nbuf)
    def _(j):
        c0 = nbuf * j
        for b in range(nbuf):
            wait_gather(c0 + b, b)
            scatter(c0 + b, b)

            @pl.when(c0 + b + nbuf < nch)
            def _():
                fire_gather(c0 + b + nbuf, b)

    tail = nch % nbuf
    for r in range(tail):
        wait_gather(nch - tail + r, r)
        scatter(nch - tail + r, r)


def _sc_agg1(x, src3, dst3):
    mesh = plsc.VectorSubcoreMesh(core_axis_name="c", subcore_axis_name="s")

    @functools.partial(
        pl.kernel,
        out_type=[jax.ShapeDtypeStruct((NC, NPAD, D), jnp.float32),
                  jax.ShapeDtypeStruct((NC, NPAD), jnp.float32)],
        mesh=mesh,
        scratch_types=[
            pltpu.VMEM((BLK1, CHUNK1), jnp.int32),
            pltpu.VMEM((BLK1, CHUNK1), jnp.int32),
            pltpu.VMEM((CHUNK1, D), jnp.float32),
            pltpu.VMEM((CHUNK1, D), jnp.float32),
            pltpu.VMEM((CHUNK1, D), jnp.float32),
            pltpu.VMEM((CHUNK1, D), jnp.float32),
            pltpu.VMEM((CHUNK1, D), jnp.float32),
            pltpu.VMEM((CHUNK1, D), jnp.float32),
            pltpu.VMEM((CHUNK1,), jnp.float32),
            pltpu.VMEM((ROWS_PER_SUB,), jnp.float32),
            pltpu.VMEM_SHARED((NPAD, D), jnp.float32),
            pltpu.VMEM_SHARED((NPAD,), jnp.float32),
            pltpu.SemaphoreType.DMA,
            pltpu.SemaphoreType.DMA,
            pltpu.SemaphoreType.DMA,
            pltpu.SemaphoreType.DMA,
            pltpu.SemaphoreType.DMA,
            pltpu.SemaphoreType.DMA,
            pltpu.SemaphoreType.DMA,
        ],
    )
    def k(x_hbm, src_hbm, dst_hbm, osum_hbm, ocnt_hbm,
          sidx, didx, rows0, rows1, rows2, rows3, rows4, rows5, ones, zcnt,
          acc, acc_cnt, sem0, sem1, sem2, sem3, sem4, sem5, isem):
        c = lax.axis_index("c")
        s = lax.axis_index("s")
        wid = c * NS + s
        rows = (rows0, rows1, rows2, rows3, rows4, rows5)
        sem = (sem0, sem1, sem2, sem3, sem4, sem5)

        pltpu.async_copy(src_hbm.at[wid, 0], sidx, isem)
        pltpu.async_copy(dst_hbm.at[wid, 0], didx, isem)

        @pl.loop(0, CHUNK1, step=16)
        def _(j):
            ones[pl.ds(j, 16)] = jnp.ones((16,), jnp.float32)

        _zero_acc_rows(rows0, acc, s, chunk=CHUNK1)

        @pl.loop(0, ROWS_PER_SUB, step=16)
        def _(j):
            zcnt[pl.ds(j, 16)] = jnp.zeros((16,), jnp.float32)

        pltpu.sync_copy(zcnt, acc_cnt.at[pl.ds(s * ROWS_PER_SUB,
                                               ROWS_PER_SUB)])

        pltpu.make_async_copy(src_hbm.at[wid, 0], sidx, isem).wait()
        pltpu.make_async_copy(dst_hbm.at[wid, 0], didx, isem).wait()

        plsc.subcore_barrier()

        def fire(ci, b):
            pltpu.async_copy(x_hbm.at[sidx.at[ci]], rows[b], sem[b])

        def wait(ci, b):
            pltpu.make_async_copy(x_hbm.at[sidx.at[ci]], rows[b],
                                  sem[b]).wait()

        def scat(ci, b):
            pltpu.sync_copy(rows[b], acc.at[didx.at[ci]], add=True)
            pltpu.sync_copy(ones, acc_cnt.at[didx.at[ci]], add=True)

        for blk in range(NBLK1):
            if blk > 0:
                pltpu.sync_copy(src_hbm.at[wid, blk], sidx)
                pltpu.sync_copy(dst_hbm.at[wid, blk], didx)
            _edge_pipeline(BLK1, fire, wait, scat, nbuf=NBUF2)

        plsc.subcore_barrier()
        r0 = s * ROWS_PER_SUB
        pltpu.sync_copy(acc.at[pl.ds(r0, ROWS_PER_SUB)],
                        osum_hbm.at[c, pl.ds(r0, ROWS_PER_SUB)])
        pltpu.sync_copy(acc_cnt.at[pl.ds(r0, ROWS_PER_SUB)],
                        ocnt_hbm.at[c, pl.ds(r0, ROWS_PER_SUB)])

    return k(x, src3, dst3)


def _sc_agg2(h0, h1, src3, dst3):
    mesh = plsc.VectorSubcoreMesh(core_axis_name="c", subcore_axis_name="s")

    @functools.partial(
        pl.kernel,
        out_type=jax.ShapeDtypeStruct((NC, NPAD, D), jnp.float32),
        mesh=mesh,
        scratch_types=[
            pltpu.VMEM((BLK2, CHUNK2), jnp.int32),
            pltpu.VMEM((BLK2, CHUNK2), jnp.int32),
            pltpu.VMEM((CHUNK2, D), jnp.float32),
            pltpu.VMEM((CHUNK2, D), jnp.float32),
            pltpu.VMEM((CHUNK2, D), jnp.float32),
            pltpu.VMEM((CHUNK2, D), jnp.float32),
            pltpu.VMEM((CHUNK2, D), jnp.float32),
            pltpu.VMEM((CHUNK2, D), jnp.float32),
            pltpu.VMEM_SHARED((NPAD, D), jnp.float32),
            pltpu.SemaphoreType.DMA,
            pltpu.SemaphoreType.DMA,
            pltpu.SemaphoreType.DMA,
            pltpu.SemaphoreType.DMA,
            pltpu.SemaphoreType.DMA,
            pltpu.SemaphoreType.DMA,
            pltpu.SemaphoreType.DMA,
        ],
    )
    def k(h0_hbm, h1_hbm, src_hbm, dst_hbm, out_hbm,
          sidx, didx, rows0, rows1, rows2, rows3, rows4, rows5, acc,
          sem0, sem1, sem2, sem3, sem4, sem5, isem):
        c = lax.axis_index("c")
        s = lax.axis_index("s")
        rows = (rows0, rows1, rows2, rows3, rows4, rows5)
        sem = (sem0, sem1, sem2, sem3, sem4, sem5)

        pltpu.async_copy(src_hbm.at[s, 0], sidx, isem)
        pltpu.async_copy(dst_hbm.at[s, 0], didx, isem)
        _zero_acc_rows(rows0, acc, s, chunk=CHUNK2)
        pltpu.make_async_copy(src_hbm.at[s, 0], sidx, isem).wait()
        pltpu.make_async_copy(dst_hbm.at[s, 0], didx, isem).wait()
        plsc.subcore_barrier()

        def scat(ci, b):
            pltpu.sync_copy(rows[b], acc.at[didx.at[ci]], add=True)

        for half in range(NBLK2):
            if half > 0:
                pltpu.sync_copy(src_hbm.at[s, half], sidx)
                pltpu.sync_copy(dst_hbm.at[s, half], didx)

            @pl.when(c == 0)
            def _():
                def fire(ci, b):
                    pltpu.async_copy(h0_hbm.at[sidx.at[ci]], rows[b], sem[b])

                def wait(ci, b):
                    pltpu.make_async_copy(h0_hbm.at[sidx.at[ci]], rows[b],
                                          sem[b]).wait()

                _edge_pipeline(BLK2, fire, wait, scat, nbuf=NBUF2)

            @pl.when(c == 1)
            def _():
                def fire(ci, b):
                    pltpu.async_copy(h1_hbm.at[sidx.at[ci]], rows[b], sem[b])

                def wait(ci, b):
                    pltpu.make_async_copy(h1_hbm.at[sidx.at[ci]], rows[b],
                                          sem[b]).wait()

                _edge_pipeline(BLK2, fire, wait, scat, nbuf=NBUF2)

        plsc.subcore_barrier()
        r0 = s * ROWS_PER_SUB
        pltpu.sync_copy(acc.at[pl.ds(r0, ROWS_PER_SUB)],
                        out_hbm.at[c, pl.ds(r0, ROWS_PER_SUB)])

    return k(h0, h1, src3, dst3)


def _tc_layer1(sum1, cnt, x, W1l, b1, W1r):
    def body(a_ref, c_ref, x_ref, wl_ref, b_ref, wr_ref, h0_ref, h1_ref):
        ssum = a_ref[0] + a_ref[1]
        deg = c_ref[0] + c_ref[1]
        mean = ssum / jnp.clip(deg, 1.0)[:, None]
        h = jnp.dot(mean, wl_ref[...].T, preferred_element_type=jnp.float32)
        h = h + jnp.dot(x_ref[...], wr_ref[...].T,
                        preferred_element_type=jnp.float32)
        h = jnp.maximum(h + b_ref[...], 0.0)
        h0_ref[...] = h[:, :D]
        h1_ref[...] = h[:, D:]

    return pl.pallas_call(
        body,
        grid=(pl.cdiv(N, RB),),
        in_specs=[
            pl.BlockSpec((NC, RB, D), lambda i: (0, i, 0)),
            pl.BlockSpec((NC, RB), lambda i: (0, i)),
            pl.BlockSpec((RB, D), lambda i: (i, 0)),
            pl.BlockSpec((H, D), lambda i: (0, 0)),
            pl.BlockSpec((1, H), lambda i: (0, 0)),
            pl.BlockSpec((H, D), lambda i: (0, 0)),
        ],
        out_specs=[
            pl.BlockSpec((RB, D), lambda i: (i, 0)),
            pl.BlockSpec((RB, D), lambda i: (i, 0)),
        ],
        out_shape=[jax.ShapeDtypeStruct((N, D), jnp.float32),
                   jax.ShapeDtypeStruct((N, D), jnp.float32)],
    )(sum1, cnt, x, W1l, b1.reshape(1, H), W1r)


def _tc_layer2(agg2, cnt, h0, h1, W2l, b2, W2r):
    def body(g_ref, c_ref, h0_ref, h1_ref, wl_ref, b_ref, wr_ref, o_ref):
        deg = c_ref[0] + c_ref[1]
        inv = 1.0 / jnp.clip(deg, 1.0)
        m0 = g_ref[0] * inv[:, None]
        m1 = g_ref[1] * inv[:, None]
        wl = wl_ref[...]
        wr = wr_ref[...]
        o = jnp.dot(m0, wl[:, :D].T, preferred_element_type=jnp.float32)
        o = o + jnp.dot(m1, wl[:, D:].T, preferred_element_type=jnp.float32)
        o = o + jnp.dot(h0_ref[...], wr[:, :D].T,
                        preferred_element_type=jnp.float32)
        o = o + jnp.dot(h1_ref[...], wr[:, D:].T,
                        preferred_element_type=jnp.float32)
        o_ref[...] = o + b_ref[...]

    return pl.pallas_call(
        body,
        grid=(pl.cdiv(N, RB),),
        in_specs=[
            pl.BlockSpec((NC, RB, D), lambda i: (0, i, 0)),
            pl.BlockSpec((NC, RB), lambda i: (0, i)),
            pl.BlockSpec((RB, D), lambda i: (i, 0)),
            pl.BlockSpec((RB, D), lambda i: (i, 0)),
            pl.BlockSpec((H, H), lambda i: (0, 0)),
            pl.BlockSpec((1, H), lambda i: (0, 0)),
            pl.BlockSpec((H, H), lambda i: (0, 0)),
        ],
        out_specs=pl.BlockSpec((RB, H), lambda i: (i, 0)),
        out_shape=jax.ShapeDtypeStruct((N, H), jnp.float32),
    )(agg2, cnt, h0, h1, W2l, b2.reshape(1, H), W2r)


def kernel(x, edge_index, W1l, b1, W1r, W2l, b2, W2r):
    ei = edge_index.astype(jnp.int32)
    src = ei[0]
    dst = ei[1]
    src3a = src.reshape(NC * NS, NBLK1, BLK1, CHUNK1)
    dst3a = dst.reshape(NC * NS, NBLK1, BLK1, CHUNK1)
    src3b = src.reshape(NS, NBLK2, BLK2, CHUNK2)
    dst3b = dst.reshape(NS, NBLK2, BLK2, CHUNK2)
    sum1, cnt = _sc_agg1(x, src3a, dst3a)
    h0, h1 = _tc_layer1(sum1, cnt, x, W1l, b1, W1r)
    agg2 = _sc_agg2(h0, h1, src3b, dst3b)
    return _tc_layer2(agg2, cnt, h0, h1, W2l, b2, W2r)


# speedup vs baseline: 1.0721x; 1.0721x over previous
import functools

import jax
import jax.numpy as jnp
from jax import lax
from jax.experimental import pallas as pl
from jax.experimental.pallas import tpu as pltpu
from jax.experimental.pallas import tpu_sc as plsc

N = 10000
E = 320000
D = 128
H = 256
NC = 2
NS = 16
CHUNK = 80
NPAD = 10240
ROWS_PER_SUB = NPAD // NS
ZCH = 128
RB = 1280

CHUNK1 = CHUNK
NCH1 = E // (NC * NS) // CHUNK1
NCH2 = E // NS // CHUNK
NBLK1 = 5
BLK1 = NCH1 // NBLK1
CHUNK2 = 40
NCH2B = E // NS // CHUNK2
NBLK2 = 10
BLK2 = NCH2B // NBLK2


def _zero_acc_rows(zrows, acc, s, chunk=CHUNK):
    @pl.loop(0, chunk)
    def _(r):
        @pl.loop(0, D, step=16)
        def _(j):
            zrows[r, pl.ds(j, 16)] = jnp.zeros((16,), jnp.float32)

    @pl.loop(0, ROWS_PER_SUB // chunk)
    def _(j):
        pltpu.sync_copy(zrows,
                        acc.at[pl.ds(s * ROWS_PER_SUB + j * chunk, chunk)])


NBUF = 3
NBUF2 = 6


def _edge_pipeline(nch, fire_gather, wait_gather, scatter, nbuf=NBUF):
    for b in range(nbuf):
        fire_gather(b, b)

    @pl.loop(0, nch // nbuf)
    def _(j):
        c0 = nbuf * j
        for b in range(nbuf):
            wait_gather(c0 + b, b)
            scatter(c0 + b, b)

            @pl.when(c0 + b + nbuf < nch)
            def _():
                fire_gather(c0 + b + nbuf, b)

    tail = nch % nbuf
    for r in range(tail):
        wait_gather(nch - tail + r, r)
        scatter(nch - tail + r, r)


def _sc_agg1(x, src3, dst3):
    mesh = plsc.VectorSubcoreMesh(core_axis_name="c", subcore_axis_name="s")

    @functools.partial(
        pl.kernel,
        out_type=[jax.ShapeDtypeStruct((NC, NPAD, D), jnp.float32),
                  jax.ShapeDtypeStruct((NC, NPAD), jnp.float32)],
        mesh=mesh,
        scratch_types=[
            pltpu.VMEM((BLK1, CHUNK1), jnp.int32),
            pltpu.VMEM((BLK1, CHUNK1), jnp.int32),
            pltpu.VMEM((CHUNK1, D), jnp.float32),
            pltpu.VMEM((CHUNK1, D), jnp.float32),
            pltpu.VMEM((CHUNK1, D), jnp.float32),
            pltpu.VMEM((CHUNK1,), jnp.float32),
            pltpu.VMEM((ROWS_PER_SUB,), jnp.float32),
            pltpu.VMEM_SHARED((NPAD, D), jnp.float32),
            pltpu.VMEM_SHARED((NPAD,), jnp.float32),
            pltpu.SemaphoreType.DMA,
            pltpu.SemaphoreType.DMA,
            pltpu.SemaphoreType.DMA,
            pltpu.SemaphoreType.DMA,
        ],
    )
    def k(x_hbm, src_hbm, dst_hbm, osum_hbm, ocnt_hbm,
          sidx, didx, rows0, rows1, rows2, ones, zcnt,
          acc, acc_cnt, sem0, sem1, sem2, isem):
        c = lax.axis_index("c")
        s = lax.axis_index("s")
        wid = c * NS + s
        rows = (rows0, rows1, rows2)
        sem = (sem0, sem1, sem2)

        pltpu.async_copy(src_hbm.at[wid, 0], sidx, isem)
        pltpu.async_copy(dst_hbm.at[wid, 0], didx, isem)

        @pl.loop(0, CHUNK1, step=16)
        def _(j):
            ones[pl.ds(j, 16)] = jnp.ones((16,), jnp.float32)

        _zero_acc_rows(rows0, acc, s, chunk=CHUNK1)

        @pl.loop(0, ROWS_PER_SUB, step=16)
        def _(j):
            zcnt[pl.ds(j, 16)] = jnp.zeros((16,), jnp.float32)

        pltpu.sync_copy(zcnt, acc_cnt.at[pl.ds(s * ROWS_PER_SUB,
                                               ROWS_PER_SUB)])

        pltpu.make_async_copy(src_hbm.at[wid, 0], sidx, isem).wait()
        pltpu.make_async_copy(dst_hbm.at[wid, 0], didx, isem).wait()

        plsc.subcore_barrier()

        def fire(ci, b):
            pltpu.async_copy(x_hbm.at[sidx.at[ci]], rows[b], sem[b])

        def wait(ci, b):
            pltpu.make_async_copy(x_hbm.at[sidx.at[ci]], rows[b],
                                  sem[b]).wait()

        def scat(ci, b):
            pltpu.sync_copy(rows[b], acc.at[didx.at[ci]], add=True)
            pltpu.sync_copy(ones, acc_cnt.at[didx.at[ci]], add=True)

        for blk in range(NBLK1):
            if blk > 0:
                pltpu.sync_copy(src_hbm.at[wid, blk], sidx)
                pltpu.sync_copy(dst_hbm.at[wid, blk], didx)
            _edge_pipeline(BLK1, fire, wait, scat)

        plsc.subcore_barrier()
        r0 = s * ROWS_PER_SUB
        pltpu.sync_copy(acc.at[pl.ds(r0, ROWS_PER_SUB)],
                        osum_hbm.at[c, pl.ds(r0, ROWS_PER_SUB)])
        pltpu.sync_copy(acc_cnt.at[pl.ds(r0, ROWS_PER_SUB)],
                        ocnt_hbm.at[c, pl.ds(r0, ROWS_PER_SUB)])

    return k(x, src3, dst3)


def _sc_agg2(h0, h1, src3, dst3):
    mesh = plsc.VectorSubcoreMesh(core_axis_name="c", subcore_axis_name="s")

    @functools.partial(
        pl.kernel,
        out_type=jax.ShapeDtypeStruct((NC, NPAD, D), jnp.float32),
        mesh=mesh,
        scratch_types=[
            pltpu.VMEM((BLK2, CHUNK2), jnp.int32),
            pltpu.VMEM((BLK2, CHUNK2), jnp.int32),
            pltpu.VMEM((CHUNK2, D), jnp.float32),
            pltpu.VMEM((CHUNK2, D), jnp.float32),
            pltpu.VMEM((CHUNK2, D), jnp.float32),
            pltpu.VMEM((CHUNK2, D), jnp.float32),
            pltpu.VMEM((CHUNK2, D), jnp.float32),
            pltpu.VMEM((CHUNK2, D), jnp.float32),
            pltpu.VMEM_SHARED((NPAD, D), jnp.float32),
            pltpu.SemaphoreType.DMA,
            pltpu.SemaphoreType.DMA,
            pltpu.SemaphoreType.DMA,
            pltpu.SemaphoreType.DMA,
            pltpu.SemaphoreType.DMA,
            pltpu.SemaphoreType.DMA,
            pltpu.SemaphoreType.DMA,
        ],
    )
    def k(h0_hbm, h1_hbm, src_hbm, dst_hbm, out_hbm,
          sidx, didx, rows0, rows1, rows2, rows3, rows4, rows5, acc,
          sem0, sem1, sem2, sem3, sem4, sem5, isem):
        c = lax.axis_index("c")
        s = lax.axis_index("s")
        rows = (rows0, rows1, rows2, rows3, rows4, rows5)
        sem = (sem0, sem1, sem2, sem3, sem4, sem5)

        pltpu.async_copy(src_hbm.at[s, 0], sidx, isem)
        pltpu.async_copy(dst_hbm.at[s, 0], didx, isem)
        _zero_acc_rows(rows0, acc, s, chunk=CHUNK2)
        pltpu.make_async_copy(src_hbm.at[s, 0], sidx, isem).wait()
        pltpu.make_async_copy(dst_hbm.at[s, 0], didx, isem).wait()
        plsc.subcore_barrier()

        def scat(ci, b):
            pltpu.sync_copy(rows[b], acc.at[didx.at[ci]], add=True)

        for half in range(NBLK2):
            if half > 0:
                pltpu.sync_copy(src_hbm.at[s, half], sidx)
                pltpu.sync_copy(dst_hbm.at[s, half], didx)

            @pl.when(c == 0)
            def _():
                def fire(ci, b):
                    pltpu.async_copy(h0_hbm.at[sidx.at[ci]], rows[b], sem[b])

                def wait(ci, b):
                    pltpu.make_async_copy(h0_hbm.at[sidx.at[ci]], rows[b],
                                          sem[b]).wait()

                _edge_pipeline(BLK2, fire, wait, scat, nbuf=NBUF2)

            @pl.when(c == 1)
            def _():
                def fire(ci, b):
                    pltpu.async_copy(h1_hbm.at[sidx.at[ci]], rows[b], sem[b])

                def wait(ci, b):
                    pltpu.make_async_copy(h1_hbm.at[sidx.at[ci]], rows[b],
                                          sem[b]).wait()

                _edge_pipeline(BLK2, fire, wait, scat, nbuf=NBUF2)

        plsc.subcore_barrier()
        r0 = s * ROWS_PER_SUB
        pltpu.sync_copy(acc.at[pl.ds(r0, ROWS_PER_SUB)],
                        out_hbm.at[c, pl.ds(r0, ROWS_PER_SUB)])

    return k(h0, h1, src3, dst3)


def _tc_layer1(sum1, cnt, x, W1l, b1, W1r):
    def body(a_ref, c_ref, x_ref, wl_ref, b_ref, wr_ref, h0_ref, h1_ref):
        ssum = a_ref[0] + a_ref[1]
        deg = c_ref[0] + c_ref[1]
        mean = ssum / jnp.clip(deg, 1.0)[:, None]
        h = jnp.dot(mean, wl_ref[...].T, preferred_element_type=jnp.float32)
        h = h + jnp.dot(x_ref[...], wr_ref[...].T,
                        preferred_element_type=jnp.float32)
        h = jnp.maximum(h + b_ref[...], 0.0)
        h0_ref[...] = h[:, :D]
        h1_ref[...] = h[:, D:]

    return pl.pallas_call(
        body,
        grid=(pl.cdiv(N, RB),),
        in_specs=[
            pl.BlockSpec((NC, RB, D), lambda i: (0, i, 0)),
            pl.BlockSpec((NC, RB), lambda i: (0, i)),
            pl.BlockSpec((RB, D), lambda i: (i, 0)),
            pl.BlockSpec((H, D), lambda i: (0, 0)),
            pl.BlockSpec((1, H), lambda i: (0, 0)),
            pl.BlockSpec((H, D), lambda i: (0, 0)),
        ],
        out_specs=[
            pl.BlockSpec((RB, D), lambda i: (i, 0)),
            pl.BlockSpec((RB, D), lambda i: (i, 0)),
        ],
        out_shape=[jax.ShapeDtypeStruct((N, D), jnp.float32),
                   jax.ShapeDtypeStruct((N, D), jnp.float32)],
    )(sum1, cnt, x, W1l, b1.reshape(1, H), W1r)


def _tc_layer2(agg2, cnt, h0, h1, W2l, b2, W2r):
    def body(g_ref, c_ref, h0_ref, h1_ref, wl_ref, b_ref, wr_ref, o_ref):
        deg = c_ref[0] + c_ref[1]
        inv = 1.0 / jnp.clip(deg, 1.0)
        m0 = g_ref[0] * inv[:, None]
        m1 = g_ref[1] * inv[:, None]
        wl = wl_ref[...]
        wr = wr_ref[...]
        o = jnp.dot(m0, wl[:, :D].T, preferred_element_type=jnp.float32)
        o = o + jnp.dot(m1, wl[:, D:].T, preferred_element_type=jnp.float32)
        o = o + jnp.dot(h0_ref[...], wr[:, :D].T,
                        preferred_element_type=jnp.float32)
        o = o + jnp.dot(h1_ref[...], wr[:, D:].T,
                        preferred_element_type=jnp.float32)
        o_ref[...] = o + b_ref[...]

    return pl.pallas_call(
        body,
        grid=(pl.cdiv(N, RB),),
        in_specs=[
            pl.BlockSpec((NC, RB, D), lambda i: (0, i, 0)),
            pl.BlockSpec((NC, RB), lambda i: (0, i)),
            pl.BlockSpec((RB, D), lambda i: (i, 0)),
            pl.BlockSpec((RB, D), lambda i: (i, 0)),
            pl.BlockSpec((H, H), lambda i: (0, 0)),
            pl.BlockSpec((1, H), lambda i: (0, 0)),
            pl.BlockSpec((H, H), lambda i: (0, 0)),
        ],
        out_specs=pl.BlockSpec((RB, H), lambda i: (i, 0)),
        out_shape=jax.ShapeDtypeStruct((N, H), jnp.float32),
    )(agg2, cnt, h0, h1, W2l, b2.reshape(1, H), W2r)


def kernel(x, edge_index, W1l, b1, W1r, W2l, b2, W2r):
    ei = edge_index.astype(jnp.int32)
    src = ei[0]
    dst = ei[1]
    src3a = src.reshape(NC * NS, NBLK1, BLK1, CHUNK1)
    dst3a = dst.reshape(NC * NS, NBLK1, BLK1, CHUNK1)
    src3b = src.reshape(NS, NBLK2, BLK2, CHUNK2)
    dst3b = dst.reshape(NS, NBLK2, BLK2, CHUNK2)
    sum1, cnt = _sc_agg1(x, src3a, dst3a)
    h0, h1 = _tc_layer1(sum1, cnt, x, W1l, b1, W1r)
    agg2 = _sc_agg2(h0, h1, src3b, dst3b)
    return _tc_layer2(agg2, cnt, h0, h1, W2l, b2, W2r)
